# baseline (speedup 1.0000x reference)
import jax
import jax.numpy as jnp
from jax import lax
from jax.experimental import pallas as pl
from jax.experimental.pallas import tpu as pltpu
from jax.experimental.pallas import tpu_sc as plsc

_D = 1024
_B = 4
_S = 8192

_NC = 2
_NS = 16
_NW = _NC * _NS
_N = _B * _S
_PP = _S // _NW
_G = 8
_NSTEP = _PP // _G
_ROWS = _B * _G
_LANES = 16


def _body(ids_hbm, mask_hbm, wte_hbm, wpe_hbm, out_hbm, am_hbm,
          idx_v, a0, a1, a2, w0, w1, mbuf, sidx, sem_g, sem_o, sem_w):
    wid = lax.axis_index("s") * _NC + lax.axis_index("c")
    pos0 = wid * _PP
    abuf = (a0, a1, a2)
    wbuf = (w0, w1)

    for b in range(_B):
        pltpu.sync_copy(ids_hbm.at[pl.ds(b * _S + pos0, _PP)],
                        idx_v.at[pl.ds(b * _PP, _PP)])

    lanes = jnp.arange(_LANES, dtype=jnp.int32)

    def sidx_step(s, carry):
        e = s * _LANES + lanes
        val = (((e & (_ROWS - 1)) >> 3) * _S + pos0
               + ((e >> 5) * _G) + (e & (_G - 1)))
        col = pl.multiple_of((s & 1) * _LANES, _LANES)
        sidx[s >> 1, pl.ds(col, _LANES)] = val
        return carry

    lax.fori_loop(0, _NSTEP * _ROWS // _LANES, sidx_step, 0)

    def gather(t):
        buf = abuf[t % 3]
        return [
            pltpu.async_copy(
                wte_hbm.at[idx_v.at[pl.ds(b * _PP + t * _G, _G)]],
                buf.at[pl.ds(b * _G, _G)], sem_g)
            for b in range(_B)
        ]

    def store(t):
        return [pltpu.async_copy(abuf[t % 3], out_hbm.at[sidx.at[t]], sem_o)]

    def load_wpe(t):
        return pltpu.async_copy(
            wpe_hbm.at[pl.ds(pos0 + t * _G, _G)], wbuf[t % 2], sem_w)

    cp_w = [load_wpe(0), load_wpe(1)]
    g_next = gather(0)

    mbase = wid * (_N // _NW)
    pltpu.sync_copy(mask_hbm.at[pl.ds(mbase, _N // _NW)], mbuf)

    @plsc.parallel_loop(0, _N // _NW, step=_LANES, unroll=4)
    def mask_step(i):
        sl = pl.ds(pl.multiple_of(i, _LANES), _LANES)
        mbuf[sl] = (1.0 - mbuf[sl]) * -10000.0

    pltpu.sync_copy(mbuf, am_hbm.at[pl.ds(mbase, _N // _NW)])

    stores = [None] * _NSTEP
    for t in range(_NSTEP):
        cp_w[t % 2].wait()
        for g in g_next:
            g.wait()
        if t >= 2:
            for s in stores[t - 2]:
                s.wait()
        if t + 1 < _NSTEP:
            g_next = gather(t + 1)

        buf = abuf[t % 3]
        wb = wbuf[t % 2]

        @plsc.parallel_loop(0, _G * _D, step=_LANES, unroll=4)
        def add_body(i):
            p = i >> 10
            sl = pl.ds(pl.multiple_of(i & (_D - 1), _LANES), _LANES)
            wv = wb[p, sl]
            for b in range(_B):
                buf[b * _G + p, sl] = buf[b * _G + p, sl] + wv

        if t + 2 < _NSTEP:
            cp_w[t % 2] = load_wpe(t + 2)
        stores[t] = store(t)

    for t in (_NSTEP - 2, _NSTEP - 1):
        for s in stores[t]:
            s.wait()


def _make_kernel():
    mesh = plsc.VectorSubcoreMesh(core_axis_name="c", subcore_axis_name="s")
    return pl.kernel(
        _body,
        out_type=(
            jax.ShapeDtypeStruct((_N, _D), jnp.float32),
            jax.ShapeDtypeStruct((_N,), jnp.float32),
        ),
        mesh=mesh,
        scratch_types=[
            pltpu.VMEM((_B * _PP,), jnp.int32),
            pltpu.VMEM((_ROWS, _D), jnp.float32),
            pltpu.VMEM((_ROWS, _D), jnp.float32),
            pltpu.VMEM((_ROWS, _D), jnp.float32),
            pltpu.VMEM((_G, _D), jnp.float32),
            pltpu.VMEM((_G, _D), jnp.float32),
            pltpu.VMEM((_N // _NW,), jnp.float32),
            pltpu.VMEM((_NSTEP, _ROWS), jnp.int32),
            pltpu.SemaphoreType.DMA,
            pltpu.SemaphoreType.DMA,
            pltpu.SemaphoreType.DMA,
        ],
    )


def kernel(input_ids, attention_mask, wte, wpe):
    b, s = input_ids.shape
    ids = input_ids.reshape(-1).astype(jnp.int32)
    maskf = attention_mask.astype(jnp.float32).reshape(-1)
    out, am = _make_kernel()(ids, maskf, wte, wpe)
    return out.reshape(b, s, _D), am.reshape(b, 1, 1, s)


# speedup vs baseline: 1.0225x; 1.0225x over previous
import jax
import jax.numpy as jnp
from jax import lax
from jax.experimental import pallas as pl
from jax.experimental.pallas import tpu as pltpu
from jax.experimental.pallas import tpu_sc as plsc

_D = 1024
_B = 4
_S = 8192

_NC = 2
_NS = 16
_NW = _NC * _NS
_N = _B * _S
_PP = _S // _NW
_G = 8
_NSTEP = _PP // _G
_ROWS = _B * _G
_LANES = 16


def _body(ids_hbm, mask_hbm, wte_hbm, wpe_hbm, out_hbm, am_hbm,
          idx_v, a0, a1, a2, w0, w1, mbuf, sem_g, sem_o, sem_w):
    wid = lax.axis_index("s") * _NC + lax.axis_index("c")
    pos0 = wid * _PP
    abuf = (a0, a1, a2)
    wbuf = (w0, w1)

    cp_ids = [
        pltpu.async_copy(ids_hbm.at[pl.ds(b * _S + pos0, _PP)],
                         idx_v.at[pl.ds(b * _PP, _PP)], sem_g)
        for b in range(_B)
    ]

    def gather(t):
        buf = abuf[t % 3]
        return [
            pltpu.async_copy(
                wte_hbm.at[idx_v.at[pl.ds(b * _PP + t * _G, _G)]],
                buf.at[pl.ds(b * _G, _G)], sem_g)
            for b in range(_B)
        ]

    def store(t):
        buf = abuf[t % 3]
        return [
            pltpu.async_copy(
                buf.at[pl.ds(b * _G, _G)],
                out_hbm.at[pl.ds(b * _S + pos0 + t * _G, _G)], sem_o)
            for b in range(_B)
        ]

    def load_wpe(t):
        return pltpu.async_copy(
            wpe_hbm.at[pl.ds(pos0 + t * _G, _G)], wbuf[t % 2], sem_w)

    cp_w = [load_wpe(0), load_wpe(1)]
    mbase = wid * (_N // _NW)
    pltpu.sync_copy(mask_hbm.at[pl.ds(mbase, _N // _NW)], mbuf)
    for cp in cp_ids:
        cp.wait()
    g_next = gather(0)


    @plsc.parallel_loop(0, _N // _NW, step=_LANES, unroll=4)
    def mask_step(i):
        sl = pl.ds(pl.multiple_of(i, _LANES), _LANES)
        mbuf[sl] = (1.0 - mbuf[sl]) * -10000.0

    pltpu.sync_copy(mbuf, am_hbm.at[pl.ds(mbase, _N // _NW)])

    stores = [None] * _NSTEP
    for t in range(_NSTEP):
        cp_w[t % 2].wait()
        for g in g_next:
            g.wait()
        if t >= 2:
            for s in stores[t - 2]:
                s.wait()
        if t + 1 < _NSTEP:
            g_next = gather(t + 1)

        buf = abuf[t % 3]
        wb = wbuf[t % 2]

        @plsc.parallel_loop(0, _G * _D, step=_LANES, unroll=8)
        def add_body(i):
            p = i >> 10
            sl = pl.ds(pl.multiple_of(i & (_D - 1), _LANES), _LANES)
            wv = wb[p, sl]
            for b in range(_B):
                buf[b * _G + p, sl] = buf[b * _G + p, sl] + wv

        if t + 2 < _NSTEP:
            cp_w[t % 2] = load_wpe(t + 2)
        stores[t] = store(t)

    for t in (_NSTEP - 2, _NSTEP - 1):
        for s in stores[t]:
            s.wait()


def _make_kernel():
    mesh = plsc.VectorSubcoreMesh(core_axis_name="c", subcore_axis_name="s")
    return pl.kernel(
        _body,
        out_type=(
            jax.ShapeDtypeStruct((_N, _D), jnp.float32),
            jax.ShapeDtypeStruct((_N,), jnp.float32),
        ),
        mesh=mesh,
        scratch_types=[
            pltpu.VMEM((_B * _PP,), jnp.int32),
            pltpu.VMEM((_ROWS, _D), jnp.float32),
            pltpu.VMEM((_ROWS, _D), jnp.float32),
            pltpu.VMEM((_ROWS, _D), jnp.float32),
            pltpu.VMEM((_G, _D), jnp.float32),
            pltpu.VMEM((_G, _D), jnp.float32),
            pltpu.VMEM((_N // _NW,), jnp.float32),
            pltpu.SemaphoreType.DMA,
            pltpu.SemaphoreType.DMA,
            pltpu.SemaphoreType.DMA,
        ],
    )


def kernel(input_ids, attention_mask, wte, wpe):
    b, s = input_ids.shape
    ids = input_ids.reshape(-1).astype(jnp.int32)
    maskf = attention_mask.astype(jnp.float32).reshape(-1)
    out, am = _make_kernel()(ids, maskf, wte, wpe)
    return out.reshape(b, s, _D), am.reshape(b, 1, 1, s)


# speedup vs baseline: 1.0735x; 1.0498x over previous
import jax
import jax.numpy as jnp
from jax import lax
from jax.experimental import pallas as pl
from jax.experimental.pallas import tpu as pltpu
from jax.experimental.pallas import tpu_sc as plsc

_D = 1024
_B = 4
_S = 8192

_NC = 2
_NS = 16
_NW = _NC * _NS
_N = _B * _S
_PP = _S // _NW
_G = 8
_NSTEP = _PP // _G
_ROWS = _B * _G
_LANES = 16


def _body(ids_hbm, mask_hbm, wte_hbm, wpe_hbm, out_hbm, am_hbm,
          idx_v, a0, a1, a2, w0, w1, mbuf, sem_g, sem_o, sem_w):
    wid = lax.axis_index("s") * _NC + lax.axis_index("c")
    pos0 = wid * _PP
    abuf = (a0, a1, a2)
    wbuf = (w0, w1)

    cp_ids = [
        pltpu.async_copy(ids_hbm.at[pl.ds(b * _S + pos0, _PP)],
                         idx_v.at[pl.ds(b * _PP, _PP)], sem_g)
        for b in range(_B)
    ]

    def _g_refs(t, slot):
        off = pl.multiple_of(t * _G, _G)
        return [
            (wte_hbm.at[idx_v.at[pl.ds(b * _PP + off, _G)]],
             abuf[slot].at[pl.ds(b * _G, _G)])
            for b in range(_B)
        ]

    def _s_refs(t, slot):
        off = pl.multiple_of(t * _G, _G)
        return [
            (abuf[slot].at[pl.ds(b * _G, _G)],
             out_hbm.at[pl.ds(b * _S + pos0 + off, _G)])
            for b in range(_B)
        ]

    def gather(t, slot):
        return [pltpu.async_copy(s, d, sem_g) for s, d in _g_refs(t, slot)]

    def wait_gather(t, slot):
        for s, d in _g_refs(t, slot):
            pltpu.make_async_copy(s, d, sem_g).wait()

    def store(t, slot):
        for s, d in _s_refs(t, slot):
            pltpu.async_copy(s, d, sem_o)

    def wait_store(t, slot):
        for s, d in _s_refs(t, slot):
            pltpu.make_async_copy(s, d, sem_o).wait()

    def load_wpe(t, slot):
        off = pl.multiple_of(t * _G, _G)
        pltpu.async_copy(wpe_hbm.at[pl.ds(pos0 + off, _G)], wbuf[slot], sem_w)

    def wait_wpe(t, slot):
        off = pl.multiple_of(t * _G, _G)
        pltpu.make_async_copy(
            wpe_hbm.at[pl.ds(pos0 + off, _G)], wbuf[slot], sem_w).wait()

    def add_group(buf, wb):
        @plsc.parallel_loop(0, _G * _D, step=_LANES, unroll=8)
        def add_body(i):
            p = i >> 10
            sl = pl.ds(pl.multiple_of(i & (_D - 1), _LANES), _LANES)
            wv = wb[p, sl]
            for b in range(_B):
                buf[b * _G + p, sl] = buf[b * _G + p, sl] + wv

    load_wpe(0, 0)
    load_wpe(1, 1)
    mbase = wid * (_N // _NW)
    pltpu.sync_copy(mask_hbm.at[pl.ds(mbase, _N // _NW)], mbuf)
    for cp in cp_ids:
        cp.wait()
    gather(0, 0)


    @plsc.parallel_loop(0, _N // _NW, step=_LANES, unroll=4)
    def mask_step(i):
        sl = pl.ds(pl.multiple_of(i, _LANES), _LANES)
        mbuf[sl] = (1.0 - mbuf[sl]) * -10000.0

    pltpu.sync_copy(mbuf, am_hbm.at[pl.ds(mbase, _N // _NW)])

    for t in (0, 1):
        wait_wpe(t, t % 2)
        wait_gather(t, t % 3)
        gather(t + 1, (t + 1) % 3)
        add_group(abuf[t % 3], wbuf[t % 2])
        load_wpe(t + 2, t % 2)
        store(t, t % 3)

    def steady(it, carry):
        t0 = 2 + it * 6
        for k in range(6):
            t = t0 + k
            sg = (2 + k) % 3
            sw = k % 2
            wait_wpe(t, sw)
            wait_gather(t, sg)
            wait_store(t - 2, (sg + 1) % 3)
            gather(jnp.minimum(t + 1, _NSTEP - 1), (sg + 1) % 3)
            add_group(abuf[sg], wbuf[sw])
            load_wpe(jnp.minimum(t + 2, _NSTEP - 1), sw)
            store(t, sg)
        return carry

    lax.fori_loop(0, (_NSTEP - 2) // 6, steady, 0)

    wait_store(_NSTEP - 2, (_NSTEP - 2) % 3)
    wait_store(_NSTEP - 1, (_NSTEP - 1) % 3)
    wait_gather(_NSTEP - 1, _NSTEP % 3)
    wait_wpe(_NSTEP - 1, 0)
    wait_wpe(_NSTEP - 1, 1)


def _make_kernel():
    mesh = plsc.VectorSubcoreMesh(core_axis_name="c", subcore_axis_name="s")
    return pl.kernel(
        _body,
        out_type=(
            jax.ShapeDtypeStruct((_N, _D), jnp.float32),
            jax.ShapeDtypeStruct((_N,), jnp.float32),
        ),
        mesh=mesh,
        scratch_types=[
            pltpu.VMEM((_B * _PP,), jnp.int32),
            pltpu.VMEM((_ROWS, _D), jnp.float32),
            pltpu.VMEM((_ROWS, _D), jnp.float32),
            pltpu.VMEM((_ROWS, _D), jnp.float32),
            pltpu.VMEM((_G, _D), jnp.float32),
            pltpu.VMEM((_G, _D), jnp.float32),
            pltpu.VMEM((_N // _NW,), jnp.float32),
            pltpu.SemaphoreType.DMA,
            pltpu.SemaphoreType.DMA,
            pltpu.SemaphoreType.DMA,
        ],
    )


def kernel(input_ids, attention_mask, wte, wpe):
    b, s = input_ids.shape
    ids = input_ids.reshape(-1).astype(jnp.int32)
    maskf = attention_mask.astype(jnp.float32).reshape(-1)
    out, am = _make_kernel()(ids, maskf, wte, wpe)
    return out.reshape(b, s, _D), am.reshape(b, 1, 1, s)
